# baseline (speedup 1.0000x reference)
import functools

import jax
import jax.numpy as jnp
from jax import lax
from jax.experimental import pallas as pl
from jax.experimental.pallas import tpu as pltpu
from jax.experimental.pallas import tpu_sc as plsc

N_NODES = 10000
N_EDGES = 320000
D = 128
NG = D // 16

NC, NS = 2, 16
NW = NC * NS
EPW = N_EDGES // NW
CHUNK = 80
NCHUNK = EPW // CHUNK


def _proj_body(x_ref, w_ref, p1_ref, p2_ref):
    xv = x_ref[...]
    wv = w_ref[...]
    dn = (((1,), (1,)), ((), ()))
    p1_ref[...] = lax.dot_general(xv, wv[:, :D], dn,
                                  preferred_element_type=jnp.float32,
                                  precision=lax.Precision.HIGHEST)
    p2_ref[...] = lax.dot_general(xv, wv[:, D:], dn,
                                  preferred_element_type=jnp.float32,
                                  precision=lax.Precision.HIGHEST)


def _node_proj(x, W):
    return pl.pallas_call(
        _proj_body,
        out_shape=[
            jax.ShapeDtypeStruct((N_NODES, D), jnp.float32),
            jax.ShapeDtypeStruct((N_NODES, D), jnp.float32),
        ],
    )(x, W)


def _edge_body(p1_hbm, p2_hbm, src_hbm, dst_hbm, w_hbm, b_hbm, out_hbm,
               srcv, dstv, wv, bufa, bufc, bv, sem1, sem2):
    wid = lax.axis_index("s") * NC + lax.axis_index("c")
    base = wid * EPW
    pltpu.sync_copy(b_hbm, bv)

    def chunk_body(ci, _):
        off = base + ci * CHUNK
        pltpu.sync_copy(src_hbm.at[pl.ds(off, CHUNK)], srcv)
        pltpu.sync_copy(dst_hbm.at[pl.ds(off, CHUNK)], dstv)
        pltpu.sync_copy(w_hbm.at[pl.ds(off, CHUNK)], wv.at[pl.ds(0, CHUNK)])
        cpa = pltpu.async_copy(p1_hbm.at[srcv], bufa, sem1)
        cpc = pltpu.async_copy(p2_hbm.at[dstv], bufc, sem2)
        cpa.wait()
        cpc.wait()

        def edge_body(e, _):
            we = wv[pl.ds(e, 16)][0]
            for j in range(NG):
                sl = pl.ds(j * 16, 16)
                v = (bufa[e, sl] + bufc[e, sl]) * we + bv[sl]
                bufa[e, sl] = 1.0 / (1.0 + jnp.exp(-v))
            return 0

        lax.fori_loop(0, CHUNK, edge_body, 0, unroll=False)
        pltpu.sync_copy(bufa, out_hbm.at[pl.ds(off, CHUNK)])
        return 0

    lax.fori_loop(0, NCHUNK, chunk_body, 0, unroll=False)


@functools.cache
def _edge_kernel():
    return functools.partial(
        pl.kernel,
        mesh=plsc.VectorSubcoreMesh(core_axis_name="c", subcore_axis_name="s"),
        out_type=jax.ShapeDtypeStruct((N_EDGES, D), jnp.float32),
        scratch_types=[
            pltpu.VMEM((CHUNK,), jnp.int32),
            pltpu.VMEM((CHUNK,), jnp.int32),
            pltpu.VMEM((CHUNK + 16,), jnp.float32),
            pltpu.VMEM((CHUNK, D), jnp.float32),
            pltpu.VMEM((CHUNK, D), jnp.float32),
            pltpu.VMEM((D,), jnp.float32),
            pltpu.SemaphoreType.DMA,
            pltpu.SemaphoreType.DMA,
        ],
    )(_edge_body)


def kernel(x, edge_index, w, W, b):
    src = edge_index[0].astype(jnp.int32)
    dst = edge_index[1].astype(jnp.int32)
    wf = w.reshape(-1)
    p1, p2 = _node_proj(x, W)
    return _edge_kernel()(p1, p2, src, dst, wf, b)


# speedup vs baseline: 6.9556x; 6.9556x over previous
import functools

import jax
import jax.numpy as jnp
from jax import lax
from jax.experimental import pallas as pl
from jax.experimental.pallas import tpu as pltpu
from jax.experimental.pallas import tpu_sc as plsc

N_NODES = 10000
N_EDGES = 320000
D = 128
NG = D // 16

NC, NS = 2, 16
NW = NC * NS
EPW = N_EDGES // NW
CHUNK = 80
NCHUNK = EPW // CHUNK
NEG_LOG2E = -1.4426950408889634


def _proj_body(x_ref, w_ref, p1_ref, p2_ref):
    xv = x_ref[...]
    wv = w_ref[...]
    dn = (((1,), (1,)), ((), ()))
    p1_ref[...] = lax.dot_general(xv, wv[:, :D], dn,
                                  preferred_element_type=jnp.float32,
                                  precision=lax.Precision.HIGHEST)
    p2_ref[...] = lax.dot_general(xv, wv[:, D:], dn,
                                  preferred_element_type=jnp.float32,
                                  precision=lax.Precision.HIGHEST)


def _node_proj(x, W):
    return pl.pallas_call(
        _proj_body,
        out_shape=[
            jax.ShapeDtypeStruct((N_NODES, D), jnp.float32),
            jax.ShapeDtypeStruct((N_NODES, D), jnp.float32),
        ],
    )(x, W)


def _edge_body(p1_hbm, p2_hbm, src_hbm, dst_hbm, w_hbm, b_hbm, out_hbm,
               srcv, dstv, wv, ga0, ga1, gc0, gc1, ob0, ob1, bv,
               sa0, sa1, sc0, sc1, so0, so1):
    ga = (ga0, ga1)
    gc = (gc0, gc1)
    ob = (ob0, ob1)
    sa = (sa0, sa1)
    sc = (sc0, sc1)
    so = (so0, so1)

    wid = lax.axis_index("s") * NC + lax.axis_index("c")
    base = wid * EPW

    pltpu.sync_copy(src_hbm.at[pl.ds(base, EPW)], srcv)
    pltpu.sync_copy(dst_hbm.at[pl.ds(base, EPW)], dstv)
    pltpu.sync_copy(w_hbm.at[pl.ds(base, EPW)], wv.at[pl.ds(0, EPW)])
    pltpu.sync_copy(b_hbm, bv)

    nb = [bv[pl.ds(j * 16, 16)] * -1.0 for j in range(NG)]

    def issue_gathers(ci, s):
        isl = pl.ds(ci * CHUNK, CHUNK)
        pltpu.async_copy(p1_hbm.at[srcv.at[isl]], ga[s], sa[s])
        pltpu.async_copy(p2_hbm.at[dstv.at[isl]], gc[s], sc[s])

    def wait_gathers(s):
        pltpu.make_async_copy(p1_hbm.at[srcv.at[pl.ds(0, CHUNK)]], ga[s], sa[s]).wait()
        pltpu.make_async_copy(p2_hbm.at[dstv.at[pl.ds(0, CHUNK)]], gc[s], sc[s]).wait()

    def wait_writeback(s):
        pltpu.make_async_copy(ob[s], out_hbm.at[pl.ds(0, CHUNK)], so[s]).wait()

    def do_chunk(ci, s, first, last):
        wait_gathers(s)
        if not first:
            wait_writeback(s)
        woff = ci * CHUNK

        @plsc.parallel_loop(0, CHUNK, unroll=4)
        def _(e):
            wl = wv[pl.ds(woff + e, 16)][0] * -1.0
            for j in range(NG):
                sl = pl.ds(j * 16, 16)
                t = (ga[s][e, sl] + gc[s][e, sl]) * wl + nb[j]
                ob[s][e, sl] = 1.0 / (1.0 + jnp.exp(t))

        pltpu.async_copy(ob[s], out_hbm.at[pl.ds(base + woff, CHUNK)], so[s])
        if not last:
            @pl.when(ci + 2 < NCHUNK)
            def _():
                issue_gathers(ci + 2, s)

    issue_gathers(0, 0)
    issue_gathers(1, 1)

    def outer(g, _):
        do_chunk(2 * g, 0, first=False, last=False)
        do_chunk(2 * g + 1, 1, first=False, last=False)
        return 0

    do_chunk(0, 0, first=True, last=False)
    do_chunk(1, 1, first=True, last=False)
    lax.fori_loop(1, NCHUNK // 2, outer, 0, unroll=False)
    do_chunk(NCHUNK - 1, 0, first=False, last=True)
    wait_writeback(1)
    wait_writeback(0)


@functools.cache
def _edge_kernel():
    return functools.partial(
        pl.kernel,
        mesh=plsc.VectorSubcoreMesh(core_axis_name="c", subcore_axis_name="s"),
        out_type=jax.ShapeDtypeStruct((N_EDGES, D), jnp.float32),
        scratch_types=[
            pltpu.VMEM((EPW,), jnp.int32),
            pltpu.VMEM((EPW,), jnp.int32),
            pltpu.VMEM((EPW + 16,), jnp.float32),
            pltpu.VMEM((CHUNK, D), jnp.float32),
            pltpu.VMEM((CHUNK, D), jnp.float32),
            pltpu.VMEM((CHUNK, D), jnp.float32),
            pltpu.VMEM((CHUNK, D), jnp.float32),
            pltpu.VMEM((CHUNK, D), jnp.float32),
            pltpu.VMEM((CHUNK, D), jnp.float32),
            pltpu.VMEM((D,), jnp.float32),
            pltpu.SemaphoreType.DMA,
            pltpu.SemaphoreType.DMA,
            pltpu.SemaphoreType.DMA,
            pltpu.SemaphoreType.DMA,
            pltpu.SemaphoreType.DMA,
            pltpu.SemaphoreType.DMA,
        ],
    )(_edge_body)


def kernel(x, edge_index, w, W, b):
    src = edge_index[0].astype(jnp.int32)
    dst = edge_index[1].astype(jnp.int32)
    wf = w.reshape(-1)
    p1, p2 = _node_proj(x, W)
    return _edge_kernel()(p1, p2, src, dst, wf, b)
